# baseline (speedup 1.0000x reference)
import jax
import jax.numpy as jnp
from jax import lax
from jax.experimental import pallas as pl
from jax.experimental.pallas import tpu as pltpu

B = 16384
C = 1000
BR = 2048
NB = B // BR
BC = 128
NC = 8


def _tc_body(yt_ref, yp_ref, out_ref, acc_ref):
    i = pl.program_id(0)
    j = pl.program_id(1)

    yt = yt_ref[0, 0, :]
    cols = j * BC + lax.broadcasted_iota(jnp.int32, (BR, BC), 1)
    mask = cols == yt[:, None]
    part = jnp.sum(jnp.where(mask, yp_ref[...], 0.0), axis=1, keepdims=True)

    @pl.when(j == 0)
    def _():
        acc_ref[...] = part

    @pl.when(j > 0)
    def _():
        acc_ref[...] += part

    @pl.when(j == NC - 1)
    def _():
        s = jnp.sum(jnp.log(acc_ref[...] + 1e-7))
        prev = jnp.where(i == 0, 0.0, out_ref[0, 0])
        total = prev + s
        out_ref[0, 0] = jnp.where(i == NB - 1, total * (-1.0 / B), total)


@jax.jit
def kernel(y_pred, y_true):
    yt = y_true.astype(jnp.int32).reshape(NB, 1, BR)
    loss = pl.pallas_call(
        _tc_body,
        grid=(NB, NC),
        in_specs=[
            pl.BlockSpec((1, 1, BR), lambda i, j: (i, 0, 0)),
            pl.BlockSpec((BR, BC), lambda i, j: (i, j)),
        ],
        out_specs=pl.BlockSpec(memory_space=pltpu.SMEM),
        out_shape=jax.ShapeDtypeStruct((1, 1), jnp.float32),
        scratch_shapes=[pltpu.VMEM((BR, 1), jnp.float32)],
    )(yt, y_pred)
    return loss[0, 0]
